# baseline (speedup 1.0000x reference)
import functools

import jax
import jax.numpy as jnp
from jax import lax
from jax.experimental import pallas as pl
from jax.experimental.pallas import tpu as pltpu
from jax.experimental.pallas import tpu_sc as plsc

N = 10000
E = 320000
D_IN = 128
D_H = 64

NC = 2
NS = 16
NW = NC * NS
CHUNK = 128
CHUNKS_PER_W = 80
E_PAD = NW * CHUNKS_PER_W * CHUNK
CA = 42
CB = 118
CMAX = max(CA, CB)
TOTAL_CHUNKS = NS * (CA + CB)
N_PAD = 10240
ROWS_PER_TILE = N_PAD // NS
DEG_W = 16


def _sc_mesh():
    return plsc.VectorSubcoreMesh(core_axis_name="c", subcore_axis_name="s")


def _deg_body(dst_hbm, ones_hbm, zeros_hbm, out_hbm, dst_v, ones_v, acc):
    c = lax.axis_index("c")
    s = lax.axis_index("s")
    wid = s * NC + c
    rows = pl.ds(s * ROWS_PER_TILE, ROWS_PER_TILE)
    pltpu.sync_copy(zeros_hbm.at[rows], acc.at[rows])
    pltpu.sync_copy(dst_hbm.at[wid], dst_v)
    pltpu.sync_copy(ones_hbm, ones_v)
    plsc.subcore_barrier()

    def step(j, carry):
        pltpu.sync_copy(ones_v, acc.at[dst_v.at[j]], add=True)
        return carry

    lax.fori_loop(0, CHUNKS_PER_W, step, 0)
    plsc.subcore_barrier()
    pltpu.sync_copy(acc.at[rows], out_hbm.at[c, rows])


_SC_PARAMS = pltpu.CompilerParams(use_tc_tiling_on_sc=False)

_deg_kernel = pl.kernel(
    _deg_body,
    out_type=jax.ShapeDtypeStruct((NC, N_PAD, DEG_W), jnp.float32),
    mesh=_sc_mesh(),
    compiler_params=_SC_PARAMS,
    scratch_types=[
        pltpu.VMEM((CHUNKS_PER_W, CHUNK), jnp.int32),
        pltpu.VMEM((CHUNK, DEG_W), jnp.float32),
        pltpu.VMEM_SHARED((N_PAD, DEG_W), jnp.float32),
    ],
)


def _agg_body(g_hbm, src_hbm, dst_hbm, zeros_hbm, out_hbm,
              src_v, dst_v, rows0_v, rows1_v, acc, sem0, sem1):
    c = lax.axis_index("c")
    s = lax.axis_index("s")
    rows = pl.ds(s * ROWS_PER_TILE, ROWS_PER_TILE)
    pltpu.sync_copy(zeros_hbm.at[rows], acc.at[rows])

    def run(start, count):
        pltpu.sync_copy(src_hbm.at[pl.ds(start, count)],
                        src_v.at[pl.ds(0, count)])
        pltpu.sync_copy(dst_hbm.at[pl.ds(start, count)],
                        dst_v.at[pl.ds(0, count)])
        plsc.subcore_barrier()

        pltpu.async_copy(g_hbm.at[src_v.at[0]], rows0_v, sem0)

        def step(t, carry):
            j0 = 2 * t
            pltpu.async_copy(g_hbm.at[src_v.at[j0 + 1]], rows1_v, sem1)
            pltpu.make_async_copy(g_hbm.at[src_v.at[j0]], rows0_v, sem0).wait()
            pltpu.sync_copy(rows0_v, acc.at[dst_v.at[j0]], add=True)

            @pl.when(t < count // 2 - 1)
            def _():
                pltpu.async_copy(g_hbm.at[src_v.at[j0 + 2]], rows0_v, sem0)

            pltpu.make_async_copy(g_hbm.at[src_v.at[j0 + 1]], rows1_v, sem1).wait()
            pltpu.sync_copy(rows1_v, acc.at[dst_v.at[j0 + 1]], add=True)
            return carry

        lax.fori_loop(0, count // 2, step, 0)

    @pl.when(c == 0)
    def _():
        run(s * CA, CA)

    @pl.when(c == 1)
    def _():
        run(NS * CA + s * CB, CB)

    plsc.subcore_barrier()
    pltpu.sync_copy(acc.at[rows], out_hbm.at[c, rows])


@functools.cache
def _agg_kernel(d):
    return pl.kernel(
        _agg_body,
        out_type=jax.ShapeDtypeStruct((NC, N_PAD, d), jnp.float32),
        mesh=_sc_mesh(),
        compiler_params=_SC_PARAMS,
        scratch_types=[
            pltpu.VMEM((CMAX, CHUNK), jnp.int32),
            pltpu.VMEM((CMAX, CHUNK), jnp.int32),
            pltpu.VMEM((CHUNK, d), jnp.float32),
            pltpu.VMEM((CHUNK, d), jnp.float32),
            pltpu.VMEM_SHARED((N_PAD, d), jnp.float32),
            pltpu.SemaphoreType.DMA,
            pltpu.SemaphoreType.DMA,
        ],
    )


ROW_BLK = 1024
GRID = N_PAD // ROW_BLK


def _stage_a_body(deg_ref, x_ref, w_ref, g_ref, dinv_ref):
    degsum = deg_ref[0, :, 0:1] + deg_ref[1, :, 0:1] + 1.0
    dinv = lax.rsqrt(degsum)
    g_ref[...] = jnp.dot(x_ref[...], w_ref[...],
                         preferred_element_type=jnp.float32) * dinv
    dinv_ref[...] = jnp.broadcast_to(dinv, (ROW_BLK, 128))


def _stage_a(deg, x, w1):
    return pl.pallas_call(
        _stage_a_body,
        grid=(GRID,),
        in_specs=[
            pl.BlockSpec((NC, ROW_BLK, DEG_W), lambda i: (0, i, 0)),
            pl.BlockSpec((ROW_BLK, D_IN), lambda i: (i, 0)),
            pl.BlockSpec((D_IN, D_H), lambda i: (0, 0)),
        ],
        out_specs=[
            pl.BlockSpec((ROW_BLK, D_H), lambda i: (i, 0)),
            pl.BlockSpec((ROW_BLK, 128), lambda i: (i, 0)),
        ],
        out_shape=[
            jax.ShapeDtypeStruct((N_PAD, D_H), jnp.float32),
            jax.ShapeDtypeStruct((N_PAD, 128), jnp.float32),
        ],
    )(deg, x, w1)


def _stage_mid_body(acc_ref, g_ref, dinv_ref, w_ref, b_ref, out_ref):
    dinv = dinv_ref[:, 0:1]
    h = acc_ref[0] + acc_ref[1] + g_ref[...]
    h = jnp.maximum(h * dinv + b_ref[...], 0.0)
    out_ref[...] = jnp.dot(h, w_ref[...],
                           preferred_element_type=jnp.float32) * dinv


def _stage_mid(acc, g, dinv, w, b, d_in, d_out):
    return pl.pallas_call(
        _stage_mid_body,
        grid=(GRID,),
        in_specs=[
            pl.BlockSpec((NC, ROW_BLK, d_in), lambda i: (0, i, 0)),
            pl.BlockSpec((ROW_BLK, d_in), lambda i: (i, 0)),
            pl.BlockSpec((ROW_BLK, 128), lambda i: (i, 0)),
            pl.BlockSpec((d_in, d_out), lambda i: (0, 0)),
            pl.BlockSpec((1, d_in), lambda i: (0, 0)),
        ],
        out_specs=pl.BlockSpec((ROW_BLK, d_out), lambda i: (i, 0)),
        out_shape=jax.ShapeDtypeStruct((N_PAD, d_out), jnp.float32),
    )(acc, g, dinv, w, b)


def _stage_final_body(acc_a_ref, acc_b_ref, g_ref, dinv_ref, b_ref, out_ref):
    dinv = dinv_ref[:, 0:1]
    ha = acc_a_ref[0] + acc_a_ref[1] + g_ref[:, :D_H]
    hb = acc_b_ref[0] + acc_b_ref[1] + g_ref[:, D_H:]
    out_ref[...] = jnp.concatenate([ha, hb], axis=1) * dinv + b_ref[...]


def _stage_final(acc_a, acc_b, g, dinv, b):
    return pl.pallas_call(
        _stage_final_body,
        grid=(GRID,),
        in_specs=[
            pl.BlockSpec((NC, ROW_BLK, D_H), lambda i: (0, i, 0)),
            pl.BlockSpec((NC, ROW_BLK, D_H), lambda i: (0, i, 0)),
            pl.BlockSpec((ROW_BLK, D_IN), lambda i: (i, 0)),
            pl.BlockSpec((ROW_BLK, 128), lambda i: (i, 0)),
            pl.BlockSpec((1, D_IN), lambda i: (0, 0)),
        ],
        out_specs=pl.BlockSpec((ROW_BLK, D_IN), lambda i: (i, 0)),
        out_shape=jax.ShapeDtypeStruct((N_PAD, D_IN), jnp.float32),
    )(acc_a, acc_b, g, dinv, b)


def kernel(x, edge_index, is_test, W1, b1, W2, b2, W3, b3):
    src = edge_index[0].astype(jnp.int32)
    dst = edge_index[1].astype(jnp.int32)
    pad = jnp.full((E_PAD - E,), N, jnp.int32)
    src2 = jnp.concatenate([src, pad]).reshape(TOTAL_CHUNKS, CHUNK)
    dst2 = jnp.concatenate([dst, pad]).reshape(TOTAL_CHUNKS, CHUNK)
    dst3 = dst2.reshape(NW, CHUNKS_PER_W, CHUNK)

    x_pad = jnp.zeros((N_PAD, D_IN), jnp.float32).at[:N].set(x)
    ones16 = jnp.ones((CHUNK, DEG_W), jnp.float32)
    zeros16 = jnp.zeros((N_PAD, DEG_W), jnp.float32)
    zeros_h = jnp.zeros((N_PAD, D_H), jnp.float32)

    agg = _agg_kernel(D_H)
    deg = _deg_kernel(dst3, ones16, zeros16)
    g1, dinv = _stage_a(deg, x_pad, W1)
    acc1 = agg(g1, src2, dst2, zeros_h)
    g2 = _stage_mid(acc1, g1, dinv, W2, b1.reshape(1, D_H), D_H, D_H)
    acc2 = agg(g2, src2, dst2, zeros_h)
    g3 = _stage_mid(acc2, g2, dinv, W3, b2.reshape(1, D_H), D_H, D_IN)
    acc3a = agg(g3[:, :D_H], src2, dst2, zeros_h)
    acc3b = agg(g3[:, D_H:], src2, dst2, zeros_h)
    out = _stage_final(acc3a, acc3b, g3, dinv, b3.reshape(1, D_IN))
    return out[:N]


# speedup vs baseline: 1.1116x; 1.1116x over previous
import functools

import jax
import jax.numpy as jnp
from jax import lax
from jax.experimental import pallas as pl
from jax.experimental.pallas import tpu as pltpu
from jax.experimental.pallas import tpu_sc as plsc

N = 10000
E = 320000
D_IN = 128
D_H = 64

NC = 2
NS = 16
NW = NC * NS
CHUNK = 128
CHUNKS_PER_W = 80
E_PAD = NW * CHUNKS_PER_W * CHUNK
CA = 118
CB = 42
CMAX = max(CA, CB)
TOTAL_CHUNKS = NS * (CA + CB)
N_PAD = 10240
ROWS_PER_TILE = N_PAD // NS
DEG_W = 16


def _sc_mesh():
    return plsc.VectorSubcoreMesh(core_axis_name="c", subcore_axis_name="s")


def _deg_body(dst_hbm, ones_hbm, zeros_hbm, out_hbm, dst_v, ones_v, acc):
    c = lax.axis_index("c")
    s = lax.axis_index("s")
    wid = s * NC + c
    rows = pl.ds(s * ROWS_PER_TILE, ROWS_PER_TILE)
    pltpu.sync_copy(zeros_hbm.at[rows], acc.at[rows])
    pltpu.sync_copy(dst_hbm.at[wid], dst_v)
    pltpu.sync_copy(ones_hbm, ones_v)
    plsc.subcore_barrier()

    def step(j, carry):
        pltpu.sync_copy(ones_v, acc.at[dst_v.at[j]], add=True)
        return carry

    lax.fori_loop(0, CHUNKS_PER_W, step, 0)
    plsc.subcore_barrier()
    pltpu.sync_copy(acc.at[rows], out_hbm.at[c, rows])


_SC_PARAMS = pltpu.CompilerParams(use_tc_tiling_on_sc=False)

_deg_kernel = pl.kernel(
    _deg_body,
    out_type=jax.ShapeDtypeStruct((NC, N_PAD, DEG_W), jnp.float32),
    mesh=_sc_mesh(),
    compiler_params=_SC_PARAMS,
    scratch_types=[
        pltpu.VMEM((CHUNKS_PER_W, CHUNK), jnp.int32),
        pltpu.VMEM((CHUNK, DEG_W), jnp.float32),
        pltpu.VMEM_SHARED((N_PAD, DEG_W), jnp.float32),
    ],
)


def _agg_body(g_hbm, src_hbm, dst_hbm, zeros_hbm, out_hbm,
              src_v, dst_v, rows0_v, rows1_v, acc, sem0, sem1):
    c = lax.axis_index("c")
    s = lax.axis_index("s")
    rows = pl.ds(s * ROWS_PER_TILE, ROWS_PER_TILE)
    pltpu.sync_copy(zeros_hbm.at[rows], acc.at[rows])

    def run(start, count):
        pltpu.sync_copy(src_hbm.at[pl.ds(start, count)],
                        src_v.at[pl.ds(0, count)])
        pltpu.sync_copy(dst_hbm.at[pl.ds(start, count)],
                        dst_v.at[pl.ds(0, count)])
        plsc.subcore_barrier()

        pltpu.async_copy(g_hbm.at[src_v.at[0]], rows0_v, sem0)

        def step(t, carry):
            j0 = 2 * t
            pltpu.async_copy(g_hbm.at[src_v.at[j0 + 1]], rows1_v, sem1)
            pltpu.make_async_copy(g_hbm.at[src_v.at[j0]], rows0_v, sem0).wait()
            pltpu.sync_copy(rows0_v, acc.at[dst_v.at[j0]], add=True)

            @pl.when(t < count // 2 - 1)
            def _():
                pltpu.async_copy(g_hbm.at[src_v.at[j0 + 2]], rows0_v, sem0)

            pltpu.make_async_copy(g_hbm.at[src_v.at[j0 + 1]], rows1_v, sem1).wait()
            pltpu.sync_copy(rows1_v, acc.at[dst_v.at[j0 + 1]], add=True)
            return carry

        lax.fori_loop(0, count // 2, step, 0)

    @pl.when(c == 0)
    def _():
        run(s * CA, CA)

    @pl.when(c == 1)
    def _():
        run(NS * CA + s * CB, CB)

    plsc.subcore_barrier()
    pltpu.sync_copy(acc.at[rows], out_hbm.at[c, rows])


@functools.cache
def _agg_kernel(d):
    return pl.kernel(
        _agg_body,
        out_type=jax.ShapeDtypeStruct((NC, N_PAD, d), jnp.float32),
        mesh=_sc_mesh(),
        compiler_params=_SC_PARAMS,
        scratch_types=[
            pltpu.VMEM((CMAX, CHUNK), jnp.int32),
            pltpu.VMEM((CMAX, CHUNK), jnp.int32),
            pltpu.VMEM((CHUNK, d), jnp.float32),
            pltpu.VMEM((CHUNK, d), jnp.float32),
            pltpu.VMEM_SHARED((N_PAD, d), jnp.float32),
            pltpu.SemaphoreType.DMA,
            pltpu.SemaphoreType.DMA,
        ],
    )


ROW_BLK = 1024
GRID = N_PAD // ROW_BLK


def _stage_a_body(deg_ref, x_ref, w_ref, g_ref, dinv_ref):
    degsum = deg_ref[0, :, 0:1] + deg_ref[1, :, 0:1] + 1.0
    dinv = lax.rsqrt(degsum)
    g_ref[...] = jnp.dot(x_ref[...], w_ref[...],
                         preferred_element_type=jnp.float32) * dinv
    dinv_ref[...] = jnp.broadcast_to(dinv, (ROW_BLK, 128))


def _stage_a(deg, x, w1):
    return pl.pallas_call(
        _stage_a_body,
        grid=(GRID,),
        in_specs=[
            pl.BlockSpec((NC, ROW_BLK, DEG_W), lambda i: (0, i, 0)),
            pl.BlockSpec((ROW_BLK, D_IN), lambda i: (i, 0)),
            pl.BlockSpec((D_IN, D_H), lambda i: (0, 0)),
        ],
        out_specs=[
            pl.BlockSpec((ROW_BLK, D_H), lambda i: (i, 0)),
            pl.BlockSpec((ROW_BLK, 128), lambda i: (i, 0)),
        ],
        out_shape=[
            jax.ShapeDtypeStruct((N_PAD, D_H), jnp.float32),
            jax.ShapeDtypeStruct((N_PAD, 128), jnp.float32),
        ],
    )(deg, x, w1)


def _stage_mid_body(acc_ref, g_ref, dinv_ref, w_ref, b_ref, out_ref):
    dinv = dinv_ref[:, 0:1]
    h = acc_ref[0] + acc_ref[1] + g_ref[...]
    h = jnp.maximum(h * dinv + b_ref[...], 0.0)
    out_ref[...] = jnp.dot(h, w_ref[...],
                           preferred_element_type=jnp.float32) * dinv


def _stage_mid(acc, g, dinv, w, b, d_in, d_out):
    return pl.pallas_call(
        _stage_mid_body,
        grid=(GRID,),
        in_specs=[
            pl.BlockSpec((NC, ROW_BLK, d_in), lambda i: (0, i, 0)),
            pl.BlockSpec((ROW_BLK, d_in), lambda i: (i, 0)),
            pl.BlockSpec((ROW_BLK, 128), lambda i: (i, 0)),
            pl.BlockSpec((d_in, d_out), lambda i: (0, 0)),
            pl.BlockSpec((1, d_in), lambda i: (0, 0)),
        ],
        out_specs=pl.BlockSpec((ROW_BLK, d_out), lambda i: (i, 0)),
        out_shape=jax.ShapeDtypeStruct((N_PAD, d_out), jnp.float32),
    )(acc, g, dinv, w, b)


def _stage_final_body(acc_a_ref, acc_b_ref, g_ref, dinv_ref, b_ref, out_ref):
    dinv = dinv_ref[:, 0:1]
    ha = acc_a_ref[0] + acc_a_ref[1] + g_ref[:, :D_H]
    hb = acc_b_ref[0] + acc_b_ref[1] + g_ref[:, D_H:]
    out_ref[...] = jnp.concatenate([ha, hb], axis=1) * dinv + b_ref[...]


def _stage_final(acc_a, acc_b, g, dinv, b):
    return pl.pallas_call(
        _stage_final_body,
        grid=(GRID,),
        in_specs=[
            pl.BlockSpec((NC, ROW_BLK, D_H), lambda i: (0, i, 0)),
            pl.BlockSpec((NC, ROW_BLK, D_H), lambda i: (0, i, 0)),
            pl.BlockSpec((ROW_BLK, D_IN), lambda i: (i, 0)),
            pl.BlockSpec((ROW_BLK, 128), lambda i: (i, 0)),
            pl.BlockSpec((1, D_IN), lambda i: (0, 0)),
        ],
        out_specs=pl.BlockSpec((ROW_BLK, D_IN), lambda i: (i, 0)),
        out_shape=jax.ShapeDtypeStruct((N_PAD, D_IN), jnp.float32),
    )(acc_a, acc_b, g, dinv, b)


def kernel(x, edge_index, is_test, W1, b1, W2, b2, W3, b3):
    src = edge_index[0].astype(jnp.int32)
    dst = edge_index[1].astype(jnp.int32)
    pad = jnp.full((E_PAD - E,), N, jnp.int32)
    src2 = jnp.concatenate([src, pad]).reshape(TOTAL_CHUNKS, CHUNK)
    dst2 = jnp.concatenate([dst, pad]).reshape(TOTAL_CHUNKS, CHUNK)
    dst3 = dst2.reshape(NW, CHUNKS_PER_W, CHUNK)

    x_pad = jnp.zeros((N_PAD, D_IN), jnp.float32).at[:N].set(x)
    ones16 = jnp.ones((CHUNK, DEG_W), jnp.float32)
    zeros16 = jnp.zeros((N_PAD, DEG_W), jnp.float32)
    zeros_h = jnp.zeros((N_PAD, D_H), jnp.float32)

    agg = _agg_kernel(D_H)
    deg = _deg_kernel(dst3, ones16, zeros16)
    g1, dinv = _stage_a(deg, x_pad, W1)
    acc1 = agg(g1, src2, dst2, zeros_h)
    g2 = _stage_mid(acc1, g1, dinv, W2, b1.reshape(1, D_H), D_H, D_H)
    acc2 = agg(g2, src2, dst2, zeros_h)
    g3 = _stage_mid(acc2, g2, dinv, W3, b2.reshape(1, D_H), D_H, D_IN)
    acc3a = agg(g3[:, :D_H], src2, dst2, zeros_h)
    acc3b = agg(g3[:, D_H:], src2, dst2, zeros_h)
    out = _stage_final(acc3a, acc3b, g3, dinv, b3.reshape(1, D_IN))
    return out[:N]


# speedup vs baseline: 2.3070x; 2.0755x over previous
import functools

import jax
import jax.numpy as jnp
from jax import lax
from jax.experimental import pallas as pl
from jax.experimental.pallas import tpu as pltpu
from jax.experimental.pallas import tpu_sc as plsc

N = 10000
E = 320000
D_IN = 128
D_H = 64

NC = 2
NS = 16
NW = NC * NS
CHUNK = 128
CHUNKS_PER_W = 80
E_PAD = NW * CHUNKS_PER_W * CHUNK
CA = 80
CB = 80
CMAX = max(CA, CB)
TOTAL_CHUNKS = NS * (CA + CB)
N_PAD = 10240
ROWS_PER_TILE = N_PAD // NS
DEG_W = 16


def _sc_mesh():
    return plsc.VectorSubcoreMesh(core_axis_name="c", subcore_axis_name="s")


def _deg_body(dst_hbm, ones_hbm, zeros_hbm, out_hbm, dst_v, ones_v, acc):
    c = lax.axis_index("c")
    s = lax.axis_index("s")
    wid = s * NC + c
    rows = pl.ds(s * ROWS_PER_TILE, ROWS_PER_TILE)
    pltpu.sync_copy(zeros_hbm.at[rows], acc.at[rows])
    pltpu.sync_copy(dst_hbm.at[wid], dst_v)
    pltpu.sync_copy(ones_hbm, ones_v)
    plsc.subcore_barrier()

    def step(j, carry):
        pltpu.sync_copy(ones_v, acc.at[dst_v.at[j]], add=True)
        return carry

    lax.fori_loop(0, CHUNKS_PER_W, step, 0)
    plsc.subcore_barrier()
    pltpu.sync_copy(acc.at[rows], out_hbm.at[c, rows])


_SC_PARAMS = pltpu.CompilerParams(use_tc_tiling_on_sc=False)

_deg_kernel = pl.kernel(
    _deg_body,
    out_type=jax.ShapeDtypeStruct((NC, N_PAD, DEG_W), jnp.float32),
    mesh=_sc_mesh(),
    compiler_params=_SC_PARAMS,
    scratch_types=[
        pltpu.VMEM((CHUNKS_PER_W, CHUNK), jnp.int32),
        pltpu.VMEM((CHUNK, DEG_W), jnp.float32),
        pltpu.VMEM_SHARED((N_PAD, DEG_W), jnp.float32),
    ],
)


def _agg_body(g_hbm, src_hbm, dst_hbm, zeros_hbm, out_hbm,
              src_v, dst_v, rows0_v, rows1_v, g_sp, acc, sem0, sem1):
    c = lax.axis_index("c")
    s = lax.axis_index("s")
    rows = pl.ds(s * ROWS_PER_TILE, ROWS_PER_TILE)
    pltpu.sync_copy(g_hbm.at[rows], g_sp.at[rows])
    pltpu.sync_copy(zeros_hbm.at[rows], acc.at[rows])

    def run(start, count):
        pltpu.sync_copy(src_hbm.at[pl.ds(start, count)],
                        src_v.at[pl.ds(0, count)])
        pltpu.sync_copy(dst_hbm.at[pl.ds(start, count)],
                        dst_v.at[pl.ds(0, count)])
        plsc.subcore_barrier()

        pltpu.async_copy(g_sp.at[src_v.at[0]], rows0_v, sem0)

        def step(t, carry):
            j0 = 2 * t
            pltpu.async_copy(g_sp.at[src_v.at[j0 + 1]], rows1_v, sem1)
            pltpu.make_async_copy(g_sp.at[src_v.at[j0]], rows0_v, sem0).wait()
            pltpu.sync_copy(rows0_v, acc.at[dst_v.at[j0]], add=True)

            @pl.when(t < count // 2 - 1)
            def _():
                pltpu.async_copy(g_sp.at[src_v.at[j0 + 2]], rows0_v, sem0)

            pltpu.make_async_copy(g_sp.at[src_v.at[j0 + 1]], rows1_v, sem1).wait()
            pltpu.sync_copy(rows1_v, acc.at[dst_v.at[j0 + 1]], add=True)
            return carry

        lax.fori_loop(0, count // 2, step, 0)

    @pl.when(c == 0)
    def _():
        run(s * CA, CA)

    @pl.when(c == 1)
    def _():
        run(NS * CA + s * CB, CB)

    plsc.subcore_barrier()
    pltpu.sync_copy(acc.at[rows], out_hbm.at[c, rows])


@functools.cache
def _agg_kernel(d):
    return pl.kernel(
        _agg_body,
        out_type=jax.ShapeDtypeStruct((NC, N_PAD, d), jnp.float32),
        mesh=_sc_mesh(),
        compiler_params=_SC_PARAMS,
        scratch_types=[
            pltpu.VMEM((CMAX, CHUNK), jnp.int32),
            pltpu.VMEM((CMAX, CHUNK), jnp.int32),
            pltpu.VMEM((CHUNK, d), jnp.float32),
            pltpu.VMEM((CHUNK, d), jnp.float32),
            pltpu.VMEM_SHARED((N_PAD, d), jnp.float32),
            pltpu.VMEM_SHARED((N_PAD, d), jnp.float32),
            pltpu.SemaphoreType.DMA,
            pltpu.SemaphoreType.DMA,
        ],
    )


ROW_BLK = 1024
GRID = N_PAD // ROW_BLK


def _stage_a_body(deg_ref, x_ref, w_ref, g_ref, dinv_ref):
    degsum = deg_ref[0, :, 0:1] + deg_ref[1, :, 0:1] + 1.0
    dinv = lax.rsqrt(degsum)
    g_ref[...] = jnp.dot(x_ref[...], w_ref[...],
                         preferred_element_type=jnp.float32) * dinv
    dinv_ref[...] = jnp.broadcast_to(dinv, (ROW_BLK, 128))


def _stage_a(deg, x, w1):
    return pl.pallas_call(
        _stage_a_body,
        grid=(GRID,),
        in_specs=[
            pl.BlockSpec((NC, ROW_BLK, DEG_W), lambda i: (0, i, 0)),
            pl.BlockSpec((ROW_BLK, D_IN), lambda i: (i, 0)),
            pl.BlockSpec((D_IN, D_H), lambda i: (0, 0)),
        ],
        out_specs=[
            pl.BlockSpec((ROW_BLK, D_H), lambda i: (i, 0)),
            pl.BlockSpec((ROW_BLK, 128), lambda i: (i, 0)),
        ],
        out_shape=[
            jax.ShapeDtypeStruct((N_PAD, D_H), jnp.float32),
            jax.ShapeDtypeStruct((N_PAD, 128), jnp.float32),
        ],
    )(deg, x, w1)


def _stage_mid_body(acc_ref, g_ref, dinv_ref, w_ref, b_ref, out_ref):
    dinv = dinv_ref[:, 0:1]
    h = acc_ref[0] + acc_ref[1] + g_ref[...]
    h = jnp.maximum(h * dinv + b_ref[...], 0.0)
    out_ref[...] = jnp.dot(h, w_ref[...],
                           preferred_element_type=jnp.float32) * dinv


def _stage_mid(acc, g, dinv, w, b, d_in, d_out):
    return pl.pallas_call(
        _stage_mid_body,
        grid=(GRID,),
        in_specs=[
            pl.BlockSpec((NC, ROW_BLK, d_in), lambda i: (0, i, 0)),
            pl.BlockSpec((ROW_BLK, d_in), lambda i: (i, 0)),
            pl.BlockSpec((ROW_BLK, 128), lambda i: (i, 0)),
            pl.BlockSpec((d_in, d_out), lambda i: (0, 0)),
            pl.BlockSpec((1, d_in), lambda i: (0, 0)),
        ],
        out_specs=pl.BlockSpec((ROW_BLK, d_out), lambda i: (i, 0)),
        out_shape=jax.ShapeDtypeStruct((N_PAD, d_out), jnp.float32),
    )(acc, g, dinv, w, b)


def _stage_final_body(acc_a_ref, acc_b_ref, g_ref, dinv_ref, b_ref, out_ref):
    dinv = dinv_ref[:, 0:1]
    ha = acc_a_ref[0] + acc_a_ref[1] + g_ref[:, :D_H]
    hb = acc_b_ref[0] + acc_b_ref[1] + g_ref[:, D_H:]
    out_ref[...] = jnp.concatenate([ha, hb], axis=1) * dinv + b_ref[...]


def _stage_final(acc_a, acc_b, g, dinv, b):
    return pl.pallas_call(
        _stage_final_body,
        grid=(GRID,),
        in_specs=[
            pl.BlockSpec((NC, ROW_BLK, D_H), lambda i: (0, i, 0)),
            pl.BlockSpec((NC, ROW_BLK, D_H), lambda i: (0, i, 0)),
            pl.BlockSpec((ROW_BLK, D_IN), lambda i: (i, 0)),
            pl.BlockSpec((ROW_BLK, 128), lambda i: (i, 0)),
            pl.BlockSpec((1, D_IN), lambda i: (0, 0)),
        ],
        out_specs=pl.BlockSpec((ROW_BLK, D_IN), lambda i: (i, 0)),
        out_shape=jax.ShapeDtypeStruct((N_PAD, D_IN), jnp.float32),
    )(acc_a, acc_b, g, dinv, b)


def kernel(x, edge_index, is_test, W1, b1, W2, b2, W3, b3):
    src = edge_index[0].astype(jnp.int32)
    dst = edge_index[1].astype(jnp.int32)
    pad = jnp.full((E_PAD - E,), N, jnp.int32)
    src2 = jnp.concatenate([src, pad]).reshape(TOTAL_CHUNKS, CHUNK)
    dst2 = jnp.concatenate([dst, pad]).reshape(TOTAL_CHUNKS, CHUNK)
    dst3 = dst2.reshape(NW, CHUNKS_PER_W, CHUNK)

    x_pad = jnp.zeros((N_PAD, D_IN), jnp.float32).at[:N].set(x)
    ones16 = jnp.ones((CHUNK, DEG_W), jnp.float32)
    zeros16 = jnp.zeros((N_PAD, DEG_W), jnp.float32)
    zeros_h = jnp.zeros((N_PAD, D_H), jnp.float32)

    agg = _agg_kernel(D_H)
    deg = _deg_kernel(dst3, ones16, zeros16)
    g1, dinv = _stage_a(deg, x_pad, W1)
    acc1 = agg(g1, src2, dst2, zeros_h)
    g2 = _stage_mid(acc1, g1, dinv, W2, b1.reshape(1, D_H), D_H, D_H)
    acc2 = agg(g2, src2, dst2, zeros_h)
    g3 = _stage_mid(acc2, g2, dinv, W3, b2.reshape(1, D_H), D_H, D_IN)
    acc3a = agg(g3[:, :D_H], src2, dst2, zeros_h)
    acc3b = agg(g3[:, D_H:], src2, dst2, zeros_h)
    out = _stage_final(acc3a, acc3b, g3, dinv, b3.reshape(1, D_IN))
    return out[:N]


# speedup vs baseline: 2.3641x; 1.0247x over previous
import functools

import jax
import jax.numpy as jnp
from jax import lax
from jax.experimental import pallas as pl
from jax.experimental.pallas import tpu as pltpu
from jax.experimental.pallas import tpu_sc as plsc

N = 10000
E = 320000
D_IN = 128
D_H = 64

NC = 2
NS = 16
NW = NC * NS
CHUNK = 128
CHUNKS_PER_W = 80
E_PAD = NW * CHUNKS_PER_W * CHUNK
CA = 80
CB = 80
CMAX = max(CA, CB)
NBUF = 2
LOOK = 1
assert CA % NBUF == 0 and CB % NBUF == 0
TOTAL_CHUNKS = NS * (CA + CB)
N_PAD = 10240
ROWS_PER_TILE = N_PAD // NS
DEG_W = 16


def _sc_mesh():
    return plsc.VectorSubcoreMesh(core_axis_name="c", subcore_axis_name="s")


def _deg_body(dst_hbm, ones_hbm, zeros_hbm, out_hbm, dst_v, ones_v, acc):
    c = lax.axis_index("c")
    s = lax.axis_index("s")
    wid = s * NC + c
    rows = pl.ds(s * ROWS_PER_TILE, ROWS_PER_TILE)
    pltpu.sync_copy(zeros_hbm.at[rows], acc.at[rows])
    pltpu.sync_copy(dst_hbm.at[wid], dst_v)
    pltpu.sync_copy(ones_hbm, ones_v)
    plsc.subcore_barrier()

    def step(j, carry):
        pltpu.sync_copy(ones_v, acc.at[dst_v.at[j]], add=True)
        return carry

    lax.fori_loop(0, CHUNKS_PER_W, step, 0)
    plsc.subcore_barrier()
    pltpu.sync_copy(acc.at[rows], out_hbm.at[c, rows])


_SC_PARAMS = pltpu.CompilerParams(use_tc_tiling_on_sc=False)

_deg_kernel = pl.kernel(
    _deg_body,
    out_type=jax.ShapeDtypeStruct((NC, N_PAD, DEG_W), jnp.float32),
    mesh=_sc_mesh(),
    compiler_params=_SC_PARAMS,
    scratch_types=[
        pltpu.VMEM((CHUNKS_PER_W, CHUNK), jnp.int32),
        pltpu.VMEM((CHUNK, DEG_W), jnp.float32),
        pltpu.VMEM_SHARED((N_PAD, DEG_W), jnp.float32),
    ],
)


def _agg_body(g_hbm, src_hbm, dst_hbm, zeros_hbm, out_hbm,
              src_v, dst_v, *rest):
    bufs = rest[:NBUF]
    g_sp, acc = rest[NBUF], rest[NBUF + 1]
    gsems = rest[NBUF + 2:2 * NBUF + 2]
    ssems = rest[2 * NBUF + 2:3 * NBUF + 2]
    psem = rest[3 * NBUF + 2]
    c = lax.axis_index("c")
    s = lax.axis_index("s")
    rows = pl.ds(s * ROWS_PER_TILE, ROWS_PER_TILE)
    pltpu.async_copy(g_hbm.at[rows], g_sp.at[rows], psem)
    pltpu.async_copy(zeros_hbm.at[rows], acc.at[rows], psem)

    def run(start, count):
        pltpu.async_copy(src_hbm.at[pl.ds(start, count)],
                         src_v.at[pl.ds(0, count)], psem)
        pltpu.async_copy(dst_hbm.at[pl.ds(start, count)],
                         dst_v.at[pl.ds(0, count)], psem)
        pltpu.make_async_copy(g_hbm.at[rows], g_sp.at[rows], psem).wait()
        pltpu.make_async_copy(zeros_hbm.at[rows], acc.at[rows], psem).wait()
        pltpu.make_async_copy(src_hbm.at[pl.ds(start, count)],
                              src_v.at[pl.ds(0, count)], psem).wait()
        pltpu.make_async_copy(dst_hbm.at[pl.ds(start, count)],
                              dst_v.at[pl.ds(0, count)], psem).wait()
        plsc.subcore_barrier()

        def gather(j, b):
            pltpu.async_copy(g_sp.at[src_v.at[j]], bufs[b], gsems[b])

        def wait_gather(j, b):
            pltpu.make_async_copy(g_sp.at[src_v.at[j]], bufs[b],
                                  gsems[b]).wait()

        def scatter(j, b):
            pltpu.async_copy(bufs[b], acc.at[dst_v.at[j]], ssems[b], add=True)

        def wait_scatter(j, b):
            pltpu.make_async_copy(bufs[b], acc.at[dst_v.at[j]],
                                  ssems[b]).wait()

        for j in range(LOOK):
            gather(j, j % NBUF)

        def step(t, carry):
            base = NBUF * t
            for b in range(NBUF):
                j = base + b
                wait_gather(j, b)
                scatter(j, b)
                jj = j + LOOK
                bb = (b + LOOK) % NBUF

                @pl.when(jj < count)
                def _(jj=jj, bb=bb):
                    @pl.when(jj >= NBUF)
                    def _():
                        wait_scatter(jj - NBUF, bb)
                    gather(jj, bb)
            return carry

        lax.fori_loop(0, count // NBUF, step, 0)
        for b in range(NBUF):
            wait_scatter(count - NBUF + b, b)

    @pl.when(c == 0)
    def _():
        run(s * CA, CA)

    @pl.when(c == 1)
    def _():
        run(NS * CA + s * CB, CB)

    plsc.subcore_barrier()
    pltpu.sync_copy(acc.at[rows], out_hbm.at[c, rows])


@functools.cache
def _agg_kernel(d):
    return pl.kernel(
        _agg_body,
        out_type=jax.ShapeDtypeStruct((NC, N_PAD, d), jnp.float32),
        mesh=_sc_mesh(),
        compiler_params=_SC_PARAMS,
        scratch_types=(
            [pltpu.VMEM((CMAX, CHUNK), jnp.int32)] * 2
            + [pltpu.VMEM((CHUNK, d), jnp.float32)] * NBUF
            + [pltpu.VMEM_SHARED((N_PAD, d), jnp.float32)] * 2
            + [pltpu.SemaphoreType.DMA] * (2 * NBUF + 1)
        ),
    )


ROW_BLK = 1024
GRID = N_PAD // ROW_BLK


def _stage_a_body(deg_ref, x_ref, w_ref, g_ref, dinv_ref):
    degsum = deg_ref[0, :, 0:1] + deg_ref[1, :, 0:1] + 1.0
    dinv = lax.rsqrt(degsum)
    g_ref[...] = jnp.dot(x_ref[...], w_ref[...],
                         preferred_element_type=jnp.float32) * dinv
    dinv_ref[...] = jnp.broadcast_to(dinv, (ROW_BLK, 128))


def _stage_a(deg, x, w1):
    return pl.pallas_call(
        _stage_a_body,
        grid=(GRID,),
        in_specs=[
            pl.BlockSpec((NC, ROW_BLK, DEG_W), lambda i: (0, i, 0)),
            pl.BlockSpec((ROW_BLK, D_IN), lambda i: (i, 0)),
            pl.BlockSpec((D_IN, D_H), lambda i: (0, 0)),
        ],
        out_specs=[
            pl.BlockSpec((ROW_BLK, D_H), lambda i: (i, 0)),
            pl.BlockSpec((ROW_BLK, 128), lambda i: (i, 0)),
        ],
        out_shape=[
            jax.ShapeDtypeStruct((N_PAD, D_H), jnp.float32),
            jax.ShapeDtypeStruct((N_PAD, 128), jnp.float32),
        ],
    )(deg, x, w1)


def _stage_mid_body(acc_ref, g_ref, dinv_ref, w_ref, b_ref, out_ref):
    dinv = dinv_ref[:, 0:1]
    h = acc_ref[0] + acc_ref[1] + g_ref[...]
    h = jnp.maximum(h * dinv + b_ref[...], 0.0)
    out_ref[...] = jnp.dot(h, w_ref[...],
                           preferred_element_type=jnp.float32) * dinv


def _stage_mid(acc, g, dinv, w, b, d_in, d_out):
    return pl.pallas_call(
        _stage_mid_body,
        grid=(GRID,),
        in_specs=[
            pl.BlockSpec((NC, ROW_BLK, d_in), lambda i: (0, i, 0)),
            pl.BlockSpec((ROW_BLK, d_in), lambda i: (i, 0)),
            pl.BlockSpec((ROW_BLK, 128), lambda i: (i, 0)),
            pl.BlockSpec((d_in, d_out), lambda i: (0, 0)),
            pl.BlockSpec((1, d_in), lambda i: (0, 0)),
        ],
        out_specs=pl.BlockSpec((ROW_BLK, d_out), lambda i: (i, 0)),
        out_shape=jax.ShapeDtypeStruct((N_PAD, d_out), jnp.float32),
    )(acc, g, dinv, w, b)


def _stage_final_body(acc_a_ref, acc_b_ref, g_ref, dinv_ref, b_ref, out_ref):
    dinv = dinv_ref[:, 0:1]
    ha = acc_a_ref[0] + acc_a_ref[1] + g_ref[:, :D_H]
    hb = acc_b_ref[0] + acc_b_ref[1] + g_ref[:, D_H:]
    out_ref[...] = jnp.concatenate([ha, hb], axis=1) * dinv + b_ref[...]


def _stage_final(acc_a, acc_b, g, dinv, b):
    return pl.pallas_call(
        _stage_final_body,
        grid=(GRID,),
        in_specs=[
            pl.BlockSpec((NC, ROW_BLK, D_H), lambda i: (0, i, 0)),
            pl.BlockSpec((NC, ROW_BLK, D_H), lambda i: (0, i, 0)),
            pl.BlockSpec((ROW_BLK, D_IN), lambda i: (i, 0)),
            pl.BlockSpec((ROW_BLK, 128), lambda i: (i, 0)),
            pl.BlockSpec((1, D_IN), lambda i: (0, 0)),
        ],
        out_specs=pl.BlockSpec((ROW_BLK, D_IN), lambda i: (i, 0)),
        out_shape=jax.ShapeDtypeStruct((N_PAD, D_IN), jnp.float32),
    )(acc_a, acc_b, g, dinv, b)


def kernel(x, edge_index, is_test, W1, b1, W2, b2, W3, b3):
    src = edge_index[0].astype(jnp.int32)
    dst = edge_index[1].astype(jnp.int32)
    pad = jnp.full((E_PAD - E,), N, jnp.int32)
    src2 = jnp.concatenate([src, pad]).reshape(TOTAL_CHUNKS, CHUNK)
    dst2 = jnp.concatenate([dst, pad]).reshape(TOTAL_CHUNKS, CHUNK)
    dst3 = dst2.reshape(NW, CHUNKS_PER_W, CHUNK)

    x_pad = jnp.zeros((N_PAD, D_IN), jnp.float32).at[:N].set(x)
    ones16 = jnp.ones((CHUNK, DEG_W), jnp.float32)
    zeros16 = jnp.zeros((N_PAD, DEG_W), jnp.float32)
    zeros_h = jnp.zeros((N_PAD, D_H), jnp.float32)

    agg = _agg_kernel(D_H)
    deg = _deg_kernel(dst3, ones16, zeros16)
    g1, dinv = _stage_a(deg, x_pad, W1)
    acc1 = agg(g1, src2, dst2, zeros_h)
    g2 = _stage_mid(acc1, g1, dinv, W2, b1.reshape(1, D_H), D_H, D_H)
    acc2 = agg(g2, src2, dst2, zeros_h)
    g3 = _stage_mid(acc2, g2, dinv, W3, b2.reshape(1, D_H), D_H, D_IN)
    acc3a = agg(g3[:, :D_H], src2, dst2, zeros_h)
    acc3b = agg(g3[:, D_H:], src2, dst2, zeros_h)
    out = _stage_final(acc3a, acc3b, g3, dinv, b3.reshape(1, D_IN))
    return out[:N]


# speedup vs baseline: 2.3801x; 1.0068x over previous
import functools

import jax
import jax.numpy as jnp
from jax import lax
from jax.experimental import pallas as pl
from jax.experimental.pallas import tpu as pltpu
from jax.experimental.pallas import tpu_sc as plsc

N = 10000
E = 320000
D_IN = 128
D_H = 64

NC = 2
NS = 16
NW = NC * NS
CHUNK = 128
CHUNKS_PER_W = 80
E_PAD = NW * CHUNKS_PER_W * CHUNK
CA = 84
CB = 76
CMAX = max(CA, CB)
NBUF = 2
LOOK = 1
assert CA % NBUF == 0 and CB % NBUF == 0
TOTAL_CHUNKS = NS * (CA + CB)
N_PAD = 10240
ROWS_PER_TILE = N_PAD // NS
DEG_W = 16


def _sc_mesh():
    return plsc.VectorSubcoreMesh(core_axis_name="c", subcore_axis_name="s")


def _deg_body(dst_hbm, ones_hbm, zeros_hbm, out_hbm, dst_v, ones_v, acc):
    c = lax.axis_index("c")
    s = lax.axis_index("s")
    wid = s * NC + c
    rows = pl.ds(s * ROWS_PER_TILE, ROWS_PER_TILE)
    pltpu.sync_copy(zeros_hbm.at[rows], acc.at[rows])
    pltpu.sync_copy(dst_hbm.at[wid], dst_v)
    pltpu.sync_copy(ones_hbm, ones_v)
    plsc.subcore_barrier()

    def step(j, carry):
        pltpu.sync_copy(ones_v, acc.at[dst_v.at[j]], add=True)
        return carry

    lax.fori_loop(0, CHUNKS_PER_W, step, 0)
    plsc.subcore_barrier()
    pltpu.sync_copy(acc.at[rows], out_hbm.at[c, rows])


_SC_PARAMS = pltpu.CompilerParams(use_tc_tiling_on_sc=False)

_deg_kernel = pl.kernel(
    _deg_body,
    out_type=jax.ShapeDtypeStruct((NC, N_PAD, DEG_W), jnp.float32),
    mesh=_sc_mesh(),
    compiler_params=_SC_PARAMS,
    scratch_types=[
        pltpu.VMEM((CHUNKS_PER_W, CHUNK), jnp.int32),
        pltpu.VMEM((CHUNK, DEG_W), jnp.float32),
        pltpu.VMEM_SHARED((N_PAD, DEG_W), jnp.float32),
    ],
)


def _agg_body(g_hbm, src_hbm, dst_hbm, zeros_hbm, out_hbm,
              src_v, dst_v, *rest):
    bufs = rest[:NBUF]
    g_sp, acc = rest[NBUF], rest[NBUF + 1]
    gsems = rest[NBUF + 2:2 * NBUF + 2]
    ssems = rest[2 * NBUF + 2:3 * NBUF + 2]
    psem = rest[3 * NBUF + 2]
    c = lax.axis_index("c")
    s = lax.axis_index("s")
    rows = pl.ds(s * ROWS_PER_TILE, ROWS_PER_TILE)
    pltpu.async_copy(g_hbm.at[rows], g_sp.at[rows], psem)
    pltpu.async_copy(zeros_hbm.at[rows], acc.at[rows], psem)

    def run(start, count):
        pltpu.async_copy(src_hbm.at[pl.ds(start, count)],
                         src_v.at[pl.ds(0, count)], psem)
        pltpu.async_copy(dst_hbm.at[pl.ds(start, count)],
                         dst_v.at[pl.ds(0, count)], psem)
        pltpu.make_async_copy(g_hbm.at[rows], g_sp.at[rows], psem).wait()
        pltpu.make_async_copy(zeros_hbm.at[rows], acc.at[rows], psem).wait()
        pltpu.make_async_copy(src_hbm.at[pl.ds(start, count)],
                              src_v.at[pl.ds(0, count)], psem).wait()
        pltpu.make_async_copy(dst_hbm.at[pl.ds(start, count)],
                              dst_v.at[pl.ds(0, count)], psem).wait()
        plsc.subcore_barrier()

        def gather(j, b):
            pltpu.async_copy(g_sp.at[src_v.at[j]], bufs[b], gsems[b])

        def wait_gather(j, b):
            pltpu.make_async_copy(g_sp.at[src_v.at[j]], bufs[b],
                                  gsems[b]).wait()

        def scatter(j, b):
            pltpu.async_copy(bufs[b], acc.at[dst_v.at[j]], ssems[b], add=True)

        def wait_scatter(j, b):
            pltpu.make_async_copy(bufs[b], acc.at[dst_v.at[j]],
                                  ssems[b]).wait()

        for j in range(LOOK):
            gather(j, j % NBUF)

        def step(t, carry):
            base = NBUF * t
            for b in range(NBUF):
                j = base + b
                wait_gather(j, b)
                scatter(j, b)
                jj = j + LOOK
                bb = (b + LOOK) % NBUF

                @pl.when(jj < count)
                def _(jj=jj, bb=bb):
                    @pl.when(jj >= NBUF)
                    def _():
                        wait_scatter(jj - NBUF, bb)
                    gather(jj, bb)
            return carry

        lax.fori_loop(0, count // NBUF, step, 0)
        for b in range(NBUF):
            wait_scatter(count - NBUF + b, b)

    @pl.when(c == 0)
    def _():
        run(s * CA, CA)

    @pl.when(c == 1)
    def _():
        run(NS * CA + s * CB, CB)

    plsc.subcore_barrier()
    pltpu.sync_copy(acc.at[rows], out_hbm.at[c, rows])


@functools.cache
def _agg_kernel(d):
    return pl.kernel(
        _agg_body,
        out_type=jax.ShapeDtypeStruct((NC, N_PAD, d), jnp.float32),
        mesh=_sc_mesh(),
        compiler_params=_SC_PARAMS,
        scratch_types=(
            [pltpu.VMEM((CMAX, CHUNK), jnp.int32)] * 2
            + [pltpu.VMEM((CHUNK, d), jnp.float32)] * NBUF
            + [pltpu.VMEM_SHARED((N_PAD, d), jnp.float32)] * 2
            + [pltpu.SemaphoreType.DMA] * (2 * NBUF + 1)
        ),
    )


ROW_BLK = 1024
GRID = N_PAD // ROW_BLK


def _stage_a_body(deg_ref, x_ref, w_ref, g_ref, dinv_ref):
    degsum = deg_ref[0, :, 0:1] + deg_ref[1, :, 0:1] + 1.0
    dinv = lax.rsqrt(degsum)
    g_ref[...] = jnp.dot(x_ref[...], w_ref[...],
                         preferred_element_type=jnp.float32) * dinv
    dinv_ref[...] = jnp.broadcast_to(dinv, (ROW_BLK, 128))


def _stage_a(deg, x, w1):
    return pl.pallas_call(
        _stage_a_body,
        grid=(GRID,),
        in_specs=[
            pl.BlockSpec((NC, ROW_BLK, DEG_W), lambda i: (0, i, 0)),
            pl.BlockSpec((ROW_BLK, D_IN), lambda i: (i, 0)),
            pl.BlockSpec((D_IN, D_H), lambda i: (0, 0)),
        ],
        out_specs=[
            pl.BlockSpec((ROW_BLK, D_H), lambda i: (i, 0)),
            pl.BlockSpec((ROW_BLK, 128), lambda i: (i, 0)),
        ],
        out_shape=[
            jax.ShapeDtypeStruct((N_PAD, D_H), jnp.float32),
            jax.ShapeDtypeStruct((N_PAD, 128), jnp.float32),
        ],
    )(deg, x, w1)


def _stage_mid_body(acc_ref, g_ref, dinv_ref, w_ref, b_ref, out_ref):
    dinv = dinv_ref[:, 0:1]
    h = acc_ref[0] + acc_ref[1] + g_ref[...]
    h = jnp.maximum(h * dinv + b_ref[...], 0.0)
    out_ref[...] = jnp.dot(h, w_ref[...],
                           preferred_element_type=jnp.float32) * dinv


def _stage_mid(acc, g, dinv, w, b, d_in, d_out):
    return pl.pallas_call(
        _stage_mid_body,
        grid=(GRID,),
        in_specs=[
            pl.BlockSpec((NC, ROW_BLK, d_in), lambda i: (0, i, 0)),
            pl.BlockSpec((ROW_BLK, d_in), lambda i: (i, 0)),
            pl.BlockSpec((ROW_BLK, 128), lambda i: (i, 0)),
            pl.BlockSpec((d_in, d_out), lambda i: (0, 0)),
            pl.BlockSpec((1, d_in), lambda i: (0, 0)),
        ],
        out_specs=pl.BlockSpec((ROW_BLK, d_out), lambda i: (i, 0)),
        out_shape=jax.ShapeDtypeStruct((N_PAD, d_out), jnp.float32),
    )(acc, g, dinv, w, b)


def _stage_final_body(acc_a_ref, acc_b_ref, g_ref, dinv_ref, b_ref, out_ref):
    dinv = dinv_ref[:, 0:1]
    ha = acc_a_ref[0] + acc_a_ref[1] + g_ref[:, :D_H]
    hb = acc_b_ref[0] + acc_b_ref[1] + g_ref[:, D_H:]
    out_ref[...] = jnp.concatenate([ha, hb], axis=1) * dinv + b_ref[...]


def _stage_final(acc_a, acc_b, g, dinv, b):
    return pl.pallas_call(
        _stage_final_body,
        grid=(GRID,),
        in_specs=[
            pl.BlockSpec((NC, ROW_BLK, D_H), lambda i: (0, i, 0)),
            pl.BlockSpec((NC, ROW_BLK, D_H), lambda i: (0, i, 0)),
            pl.BlockSpec((ROW_BLK, D_IN), lambda i: (i, 0)),
            pl.BlockSpec((ROW_BLK, 128), lambda i: (i, 0)),
            pl.BlockSpec((1, D_IN), lambda i: (0, 0)),
        ],
        out_specs=pl.BlockSpec((ROW_BLK, D_IN), lambda i: (i, 0)),
        out_shape=jax.ShapeDtypeStruct((N_PAD, D_IN), jnp.float32),
    )(acc_a, acc_b, g, dinv, b)


def kernel(x, edge_index, is_test, W1, b1, W2, b2, W3, b3):
    src = edge_index[0].astype(jnp.int32)
    dst = edge_index[1].astype(jnp.int32)
    pad = jnp.full((E_PAD - E,), N, jnp.int32)
    src2 = jnp.concatenate([src, pad]).reshape(TOTAL_CHUNKS, CHUNK)
    dst2 = jnp.concatenate([dst, pad]).reshape(TOTAL_CHUNKS, CHUNK)
    dst3 = dst2.reshape(NW, CHUNKS_PER_W, CHUNK)

    x_pad = jnp.zeros((N_PAD, D_IN), jnp.float32).at[:N].set(x)
    ones16 = jnp.ones((CHUNK, DEG_W), jnp.float32)
    zeros16 = jnp.zeros((N_PAD, DEG_W), jnp.float32)
    zeros_h = jnp.zeros((N_PAD, D_H), jnp.float32)

    agg = _agg_kernel(D_H)
    deg = _deg_kernel(dst3, ones16, zeros16)
    g1, dinv = _stage_a(deg, x_pad, W1)
    acc1 = agg(g1, src2, dst2, zeros_h)
    g2 = _stage_mid(acc1, g1, dinv, W2, b1.reshape(1, D_H), D_H, D_H)
    acc2 = agg(g2, src2, dst2, zeros_h)
    g3 = _stage_mid(acc2, g2, dinv, W3, b2.reshape(1, D_H), D_H, D_IN)
    acc3a = agg(g3[:, :D_H], src2, dst2, zeros_h)
    acc3b = agg(g3[:, D_H:], src2, dst2, zeros_h)
    out = _stage_final(acc3a, acc3b, g3, dinv, b3.reshape(1, D_IN))
    return out[:N]
